# Initial kernel scaffold; baseline (speedup 1.0000x reference)
import functools

import jax
import jax.numpy as jnp
from jax import lax
from jax.experimental import pallas as pl
from jax.experimental.pallas import tpu as pltpu
from jax.experimental.pallas import tpu_sc as plsc

N = 10000
E = 320000
D = 128
NC = 2
NS = 16
NW = NC * NS
NP = 10240
EPW = E // NW
K = 80
NSB = EPW // K
SLICE = NP // NS

_mesh = plsc.VectorSubcoreMesh(
    core_axis_name="c", subcore_axis_name="s", num_cores=NC, num_subcores=NS)

_f32 = jnp.float32
_i32 = jnp.int32


def _zero_1d(ref, nwords):
  zeros = jnp.zeros((16,), ref.dtype)

  def body(i, _):
    ref[pl.ds(i * 16, 16)] = zeros
    return 0

  lax.fori_loop(0, nwords // 16, body, 0)


@functools.partial(
    pl.kernel,
    out_type=jax.ShapeDtypeStruct((4, NP), _f32),
    mesh=_mesh,
    scratch_types=[
        pltpu.VMEM((EPW,), _i32),
        pltpu.VMEM((NP,), _f32),
        pltpu.VMEM((NP,), _f32),
        pltpu.VMEM((SLICE,), _f32),
        pltpu.VMEM((SLICE,), _f32),
        pltpu.VMEM_SHARED((2, NS, NP), _f32),
    ],
)
def _degree_kernel(edges_hbm, out_hbm, idx_v, hsrc, hdst, acc, tmp, shared):
  c = lax.axis_index("c")
  s = lax.axis_index("s")
  gid = c * NS + s

  _zero_1d(hsrc, NP)
  _zero_1d(hdst, NP)

  ones = jnp.full((16,), 1.0, _f32)
  for kind, hist in ((0, hsrc), (1, hdst)):
    pltpu.sync_copy(edges_hbm.at[kind, pl.ds(gid * EPW, EPW)], idx_v)

    def hbody(j, _, hist=hist):
      ids = idx_v[pl.ds(j * 16, 16)]
      plsc.addupdate_scatter(hist, [ids], ones)
      return 0

    lax.fori_loop(0, EPW // 16, hbody, 0)

  pltpu.sync_copy(hsrc, shared.at[0, s])
  pltpu.sync_copy(hdst, shared.at[1, s])
  plsc.subcore_barrier()

  for kind in range(2):
    _zero_1d(acc, SLICE)
    for t in range(NS):
      pltpu.sync_copy(shared.at[kind, t, pl.ds(s * SLICE, SLICE)], tmp)

      def abody(i, _):
        sl = pl.ds(i * 16, 16)
        acc[sl] = acc[sl] + tmp[sl]
        return 0

      lax.fori_loop(0, SLICE // 16, abody, 0)
    pltpu.sync_copy(acc, out_hbm.at[2 * kind + c, pl.ds(s * SLICE, SLICE)])


def _pre_body(deg_ref, feat_ref, ws_ref, norms_ref, hs_ref):
  deg = deg_ref[...]
  out_deg = jnp.maximum(deg[0:1] + deg[1:2], 1.0)
  in_deg = jnp.maximum(deg[2:3] + deg[3:4], 1.0)
  norms_ref[0:1, :] = lax.rsqrt(out_deg)
  norms_ref[1:2, :] = lax.rsqrt(in_deg)
  hs_ref[...] = lax.dot_general(
      feat_ref[...], ws_ref[...], (((1,), (1,)), ((), ())),
      preferred_element_type=_f32)


_pre_call = pl.pallas_call(
    _pre_body,
    out_shape=(
        jax.ShapeDtypeStruct((2, NP), _f32),
        jax.ShapeDtypeStruct((N, D), _f32),
    ),
)


@functools.partial(
    pl.kernel,
    out_type=jax.ShapeDtypeStruct((NC, NP, D), _f32),
    mesh=_mesh,
    scratch_types=[
        pltpu.VMEM((EPW,), _i32),
        pltpu.VMEM((NSB, K), _i32),
        pltpu.VMEM((EPW,), _f32),
        pltpu.VMEM((NP,), _f32),
        pltpu.VMEM((K, D), _f32),
        pltpu.VMEM((K, D), _f32),
        pltpu.VMEM((K,), _f32),
        pltpu.VMEM((64, D), _f32),
        pltpu.VMEM_SHARED((NP, D), _f32),
        pltpu.SemaphoreType.DMA,
        pltpu.SemaphoreType.DMA,
    ],
)
def _scatter_kernel(feat_hbm, edges_hbm, dst3_hbm, ew_hbm, onorm_hbm, out_hbm,
                    src_v, dst_v, ew_v, onorm_v, rows_a, rows_b, cbuf, zbuf,
                    acc, gsem_a, gsem_b):
  c = lax.axis_index("c")
  s = lax.axis_index("s")
  gid = c * NS + s

  pltpu.sync_copy(edges_hbm.at[0, pl.ds(gid * EPW, EPW)], src_v)
  pltpu.sync_copy(dst3_hbm.at[gid], dst_v)
  pltpu.sync_copy(ew_hbm.at[pl.ds(gid * EPW, EPW)], ew_v)
  pltpu.sync_copy(onorm_hbm, onorm_v)

  zeros16 = jnp.zeros((16,), _f32)

  def zrow(i, _):
    for v in range(D // 16):
      zbuf[i, pl.ds(v * 16, 16)] = zeros16
    return 0

  lax.fori_loop(0, 64, zrow, 0)
  for blk in range(SLICE // 64):
    pltpu.sync_copy(zbuf, acc.at[pl.ds(s * SLICE + blk * 64, 64)])
  plsc.subcore_barrier()

  def fire(i, rows, gsem):
    pltpu.async_copy(feat_hbm.at[src_v.at[pl.ds(i * K, K)]], rows, gsem)

  def wait(i, rows, gsem):
    pltpu.make_async_copy(feat_hbm.at[src_v.at[pl.ds(i * K, K)]], rows,
                          gsem).wait()

  def process(i, rows):
    for v in range(K // 16):
      sl = pl.ds(i * K + v * 16, 16)
      sv = src_v[sl]
      cbuf[pl.ds(v * 16, 16)] = ew_v[sl] * plsc.load_gather(onorm_v, [sv])

    UNR = 5

    def ebody(j, _):
      for k in range(UNR):
        e = j * UNR + k
        ce = plsc.load_gather(cbuf, [jnp.full((16,), 1, _i32) * e])
        for v in range(D // 16):
          sl = pl.ds(v * 16, 16)
          rows[e, sl] = rows[e, sl] * ce
      return 0

    lax.fori_loop(0, K // UNR, ebody, 0)

    pltpu.sync_copy(rows, acc.at[dst_v.at[i]], add=True)

  fire(0, rows_a, gsem_a)

  def pair(t, _):
    i0 = 2 * t
    fire(i0 + 1, rows_b, gsem_b)
    wait(i0, rows_a, gsem_a)
    process(i0, rows_a)
    fire(i0 + 2, rows_a, gsem_a)
    wait(i0 + 1, rows_b, gsem_b)
    process(i0 + 1, rows_b)
    return 0

  lax.fori_loop(0, (NSB - 1) // 2, pair, 0)
  wait(NSB - 1, rows_a, gsem_a)
  process(NSB - 1, rows_a)

  plsc.subcore_barrier()
  pltpu.sync_copy(acc.at[pl.ds(s * SLICE, SLICE)],
                  out_hbm.at[c, pl.ds(s * SLICE, SLICE)])


def _post_body(p_ref, norms_ref, hs_ref, w_ref, b_ref, out_ref):
  agg = p_ref[0] + p_ref[1]
  h = lax.dot_general(
      agg, w_ref[...], (((1,), (1,)), ((), ())),
      preferred_element_type=_f32) + b_ref[...]
  in_col = jnp.transpose(norms_ref[1:2, :])
  h = h * in_col
  out_ref[...] = h[:N] + hs_ref[...]


_post_call = pl.pallas_call(
    _post_body,
    out_shape=jax.ShapeDtypeStruct((N, D), _f32),
)


def kernel(feature, edge_index, e_w, snorm_n, snorm_e, W_self, W, b):
  del snorm_n, snorm_e
  ew1 = e_w.reshape(E)
  dst3 = edge_index[1].reshape(NW, NSB, K)

  deg = _degree_kernel(edge_index)
  norms, h_s = _pre_call(deg, feature, W_self)
  parts = _scatter_kernel(feature, edge_index, dst3, ew1, norms[0])
  h = _post_call(parts, norms, h_s, W, b.reshape(1, D))
  return (h, e_w)


# speedup vs baseline: 4.8377x; 4.8377x over previous
import functools

import jax
import jax.numpy as jnp
from jax import lax
from jax.experimental import pallas as pl
from jax.experimental.pallas import tpu as pltpu
from jax.experimental.pallas import tpu_sc as plsc

N = 10000
E = 320000
D = 128
NC = 2
NS = 16
NW = NC * NS
NP = 10240
EPW = E // NW
K = 16
NSB = EPW // K
SLICE = NP // NS

_mesh = plsc.VectorSubcoreMesh(
    core_axis_name="c", subcore_axis_name="s", num_cores=NC, num_subcores=NS)

_f32 = jnp.float32
_i32 = jnp.int32


def _zero_1d(ref, nwords):
  zeros = jnp.zeros((16,), ref.dtype)

  def body(i, _):
    ref[pl.ds(i * 16, 16)] = zeros
    return 0

  lax.fori_loop(0, nwords // 16, body, 0)


@functools.partial(
    pl.kernel,
    out_type=jax.ShapeDtypeStruct((4, NP), _f32),
    mesh=_mesh,
    scratch_types=[
        pltpu.VMEM((EPW,), _i32),
        pltpu.VMEM((NP,), _f32),
        pltpu.VMEM((NP,), _f32),
        pltpu.VMEM((SLICE,), _f32),
        pltpu.VMEM((SLICE,), _f32),
        pltpu.VMEM_SHARED((2, NS, NP), _f32),
    ],
    compiler_params=pltpu.CompilerParams(needs_layout_passes=False),
)
def _degree_kernel(src_hbm, dst_hbm, out_hbm, idx_v, hsrc, hdst, acc, tmp,
                   shared):
  c = lax.axis_index("c")
  s = lax.axis_index("s")
  gid = c * NS + s

  _zero_1d(hsrc, NP)
  _zero_1d(hdst, NP)

  ones = jnp.full((16,), 1.0, _f32)
  for edges_hbm, hist in ((src_hbm, hsrc), (dst_hbm, hdst)):
    pltpu.sync_copy(edges_hbm.at[pl.ds(gid * EPW, EPW)], idx_v)

    def hbody(j, _, hist=hist):
      ids = idx_v[pl.ds(j * 16, 16)]
      plsc.addupdate_scatter(hist, [ids], ones)
      return 0

    lax.fori_loop(0, EPW // 16, hbody, 0)

  pltpu.sync_copy(hsrc, shared.at[0, s])
  pltpu.sync_copy(hdst, shared.at[1, s])
  plsc.subcore_barrier()

  for kind in range(2):
    _zero_1d(acc, SLICE)
    for t in range(NS):
      pltpu.sync_copy(shared.at[kind, t, pl.ds(s * SLICE, SLICE)], tmp)

      def abody(i, _):
        sl = pl.ds(i * 16, 16)
        acc[sl] = acc[sl] + tmp[sl]
        return 0

      lax.fori_loop(0, SLICE // 16, abody, 0)
    pltpu.sync_copy(acc, out_hbm.at[2 * kind + c, pl.ds(s * SLICE, SLICE)])


def _pre_body(deg_ref, feat_ref, ws_ref, norms_ref, hs_ref):
  deg = deg_ref[...]
  out_deg = jnp.maximum(deg[0:1] + deg[1:2], 1.0)
  in_deg = jnp.maximum(deg[2:3] + deg[3:4], 1.0)
  norms_ref[0:1, :] = lax.rsqrt(out_deg)
  norms_ref[1:2, :] = lax.rsqrt(in_deg)
  hs_ref[...] = lax.dot_general(
      feat_ref[...], ws_ref[...], (((1,), (1,)), ((), ())),
      preferred_element_type=_f32)


_pre_call = pl.pallas_call(
    _pre_body,
    out_shape=(
        jax.ShapeDtypeStruct((2, NP), _f32),
        jax.ShapeDtypeStruct((N, D), _f32),
    ),
)


@functools.partial(
    pl.kernel,
    out_type=jax.ShapeDtypeStruct((NC, NP, D), _f32),
    mesh=_mesh,
    scratch_types=[
        pltpu.VMEM((EPW,), _i32),
        pltpu.VMEM((EPW,), _i32),
        pltpu.VMEM((EPW,), _f32),
        pltpu.VMEM((NP,), _f32),
        pltpu.VMEM((K, D), _f32),
        pltpu.VMEM((K, D), _f32),
        pltpu.VMEM((K + 16,), _f32),
        pltpu.VMEM_SHARED((NP, D), _f32),
        pltpu.SemaphoreType.DMA,
        pltpu.SemaphoreType.DMA,
    ],
    compiler_params=pltpu.CompilerParams(needs_layout_passes=False),
)
def _scatter_kernel(feat_hbm, src_hbm, dst_hbm, ew_hbm, onorm_hbm, out_hbm,
                    src_v, dst_v, ew_v, onorm_v, rows_a, rows_b, cbuf,
                    acc, gsem_a, gsem_b):
  c = lax.axis_index("c")
  s = lax.axis_index("s")
  gid = c * NS + s

  pltpu.sync_copy(src_hbm.at[pl.ds(gid * EPW, EPW)], src_v)
  pltpu.sync_copy(dst_hbm.at[pl.ds(gid * EPW, EPW)], dst_v)
  pltpu.sync_copy(ew_hbm.at[pl.ds(gid * EPW, EPW)], ew_v)
  pltpu.sync_copy(onorm_hbm, onorm_v)

  zeros16 = jnp.zeros((16,), _f32)

  def zrow(i, _):
    for v in range(D // 16):
      rows_a[i, pl.ds(v * 16, 16)] = zeros16
    return 0

  lax.fori_loop(0, K, zrow, 0)

  def zcopy(blk, _):
    pltpu.sync_copy(rows_a, acc.at[pl.ds(s * SLICE + blk * K, K)])
    return 0

  lax.fori_loop(0, SLICE // K, zcopy, 0)
  plsc.subcore_barrier()

  def fire(i, rows, gsem):
    sidx = src_v[pl.ds(i * K, K)]
    pltpu.async_copy(feat_hbm.at[sidx], rows, gsem)

  def wait(i, rows, gsem):
    sidx = src_v[pl.ds(i * K, K)]
    pltpu.make_async_copy(feat_hbm.at[sidx], rows, gsem).wait()

  def process(i, rows):
    sl = pl.ds(i * K, K)
    sv = src_v[sl]
    cbuf[pl.ds(16, K)] = ew_v[sl] * plsc.load_gather(onorm_v, [sv])

    for e in range(K):
      ce = plsc.load_gather(cbuf, [jnp.full((16,), 16 + e, _i32)])
      for v in range(D // 16):
        vsl = pl.ds(v * 16, 16)
        rows[e, vsl] = rows[e, vsl] * ce

    didx = dst_v[pl.ds(i * K, K)]
    pltpu.sync_copy(rows, acc.at[didx], add=True)

  fire(0, rows_a, gsem_a)

  def pair(t, _):
    i0 = 2 * t
    fire(i0 + 1, rows_b, gsem_b)
    wait(i0, rows_a, gsem_a)
    process(i0, rows_a)
    fire(i0 + 2, rows_a, gsem_a)
    wait(i0 + 1, rows_b, gsem_b)
    process(i0 + 1, rows_b)
    return 0

  lax.fori_loop(0, (NSB - 1) // 2, pair, 0)
  wait(NSB - 1, rows_a, gsem_a)
  process(NSB - 1, rows_a)

  plsc.subcore_barrier()
  pltpu.sync_copy(acc.at[pl.ds(s * SLICE, SLICE)],
                  out_hbm.at[c, pl.ds(s * SLICE, SLICE)])


def _post_body(p_ref, norms_ref, hs_ref, w_ref, b_ref, out_ref):
  agg = p_ref[0] + p_ref[1]
  h = lax.dot_general(
      agg, w_ref[...], (((1,), (1,)), ((), ())),
      preferred_element_type=_f32) + b_ref[...]
  in_col = jnp.transpose(norms_ref[1:2, :])
  h = h * in_col
  out_ref[...] = h[:N] + hs_ref[...]


_post_call = pl.pallas_call(
    _post_body,
    out_shape=jax.ShapeDtypeStruct((N, D), _f32),
)


def kernel(feature, edge_index, e_w, snorm_n, snorm_e, W_self, W, b):
  del snorm_n, snorm_e
  ew1 = e_w.reshape(E)
  src1 = edge_index[0]
  dst1 = edge_index[1]

  deg = _degree_kernel(src1, dst1)
  norms, h_s = _pre_call(deg, feature, W_self)
  parts = _scatter_kernel(feature, src1, dst1, ew1, norms[0])
  h = _post_call(parts, norms, h_s, W, b.reshape(1, D))
  return (h, e_w)


# speedup vs baseline: 7.5423x; 1.5591x over previous
import functools

import jax
import jax.numpy as jnp
from jax import lax
from jax.experimental import pallas as pl
from jax.experimental.pallas import tpu as pltpu
from jax.experimental.pallas import tpu_sc as plsc

N = 10000
E = 320000
D = 128
NC = 2
NS = 16
NW = NC * NS
NP = 10240
EPW = E // NW
K = 80
NSB = EPW // K
SLICE = NP // NS

_mesh = plsc.VectorSubcoreMesh(
    core_axis_name="c", subcore_axis_name="s", num_cores=NC, num_subcores=NS)

_f32 = jnp.float32
_i32 = jnp.int32


def _zero_1d(ref, nwords):
  zeros = jnp.zeros((16,), ref.dtype)

  def body(i, _):
    ref[pl.ds(i * 16, 16)] = zeros
    return 0

  lax.fori_loop(0, nwords // 16, body, 0)


@functools.partial(
    pl.kernel,
    out_type=jax.ShapeDtypeStruct((4, NP), _f32),
    mesh=_mesh,
    scratch_types=[
        pltpu.VMEM((EPW,), _i32),
        pltpu.VMEM((NP,), _f32),
        pltpu.VMEM((NP,), _f32),
        pltpu.VMEM((SLICE,), _f32),
        pltpu.VMEM((SLICE,), _f32),
        pltpu.VMEM_SHARED((2, NS, NP), _f32),
    ],
    compiler_params=pltpu.CompilerParams(needs_layout_passes=False),
)
def _degree_kernel(src_hbm, dst_hbm, out_hbm, idx_v, hsrc, hdst, acc, tmp,
                   shared):
  c = lax.axis_index("c")
  s = lax.axis_index("s")
  gid = c * NS + s

  _zero_1d(hsrc, NP)
  _zero_1d(hdst, NP)

  ones = jnp.full((16,), 1.0, _f32)
  for edges_hbm, hist in ((src_hbm, hsrc), (dst_hbm, hdst)):
    pltpu.sync_copy(edges_hbm.at[pl.ds(gid * EPW, EPW)], idx_v)

    def hbody(j, _, hist=hist):
      ids = idx_v[pl.ds(j * 16, 16)]
      plsc.addupdate_scatter(hist, [ids], ones)
      return 0

    lax.fori_loop(0, EPW // 16, hbody, 0)

  pltpu.sync_copy(hsrc, shared.at[0, s])
  pltpu.sync_copy(hdst, shared.at[1, s])
  plsc.subcore_barrier()

  for kind in range(2):
    _zero_1d(acc, SLICE)
    for t in range(NS):
      pltpu.sync_copy(shared.at[kind, t, pl.ds(s * SLICE, SLICE)], tmp)

      def abody(i, _):
        sl = pl.ds(i * 16, 16)
        acc[sl] = acc[sl] + tmp[sl]
        return 0

      lax.fori_loop(0, SLICE // 16, abody, 0)
    pltpu.sync_copy(acc, out_hbm.at[2 * kind + c, pl.ds(s * SLICE, SLICE)])


def _pre_body(deg_ref, feat_ref, ws_ref, norms_ref, hs_ref):
  deg = deg_ref[...]
  out_deg = jnp.maximum(deg[0:1] + deg[1:2], 1.0)
  in_deg = jnp.maximum(deg[2:3] + deg[3:4], 1.0)
  norms_ref[0:1, :] = lax.rsqrt(out_deg)
  norms_ref[1:2, :] = lax.rsqrt(in_deg)
  hs_ref[...] = lax.dot_general(
      feat_ref[...], ws_ref[...], (((1,), (1,)), ((), ())),
      preferred_element_type=_f32)


_pre_call = pl.pallas_call(
    _pre_body,
    out_shape=(
        jax.ShapeDtypeStruct((2, NP), _f32),
        jax.ShapeDtypeStruct((N, D), _f32),
    ),
)


@functools.partial(
    pl.kernel,
    out_type=jax.ShapeDtypeStruct((E,), _f32),
    mesh=_mesh,
    scratch_types=[
        pltpu.VMEM((EPW,), _i32),
        pltpu.VMEM((EPW,), _f32),
        pltpu.VMEM((NP,), _f32),
    ],
    compiler_params=pltpu.CompilerParams(needs_layout_passes=False),
)
def _coef_kernel(src_hbm, ew_hbm, onorm_hbm, out_hbm, src_v, ew_v, onorm_v):
  c = lax.axis_index("c")
  s = lax.axis_index("s")
  gid = c * NS + s
  pltpu.sync_copy(src_hbm.at[pl.ds(gid * EPW, EPW)], src_v)
  pltpu.sync_copy(ew_hbm.at[pl.ds(gid * EPW, EPW)], ew_v)
  pltpu.sync_copy(onorm_hbm, onorm_v)

  def body(j, _):
    sl = pl.ds(j * 16, 16)
    ew_v[sl] = ew_v[sl] * plsc.load_gather(onorm_v, [src_v[sl]])
    return 0

  lax.fori_loop(0, EPW // 16, body, 0)
  pltpu.sync_copy(ew_v, out_hbm.at[pl.ds(gid * EPW, EPW)])


@functools.partial(
    pl.kernel,
    out_type=jax.ShapeDtypeStruct((NC, NP, D), _f32),
    mesh=_mesh,
    scratch_types=[
        pltpu.VMEM((EPW,), _i32),
        pltpu.VMEM((EPW,), _f32),
        pltpu.VMEM((K, D), _f32),
        pltpu.VMEM((K, D), _f32),
        pltpu.VMEM((K,), _i32),
        pltpu.VMEM((K,), _i32),
        pltpu.VMEM_SHARED((NP, D), _f32),
        pltpu.SemaphoreType.DMA,
        pltpu.SemaphoreType.DMA,
        pltpu.SemaphoreType.DMA,
        pltpu.SemaphoreType.DMA,
        pltpu.SemaphoreType.DMA,
        pltpu.SemaphoreType.DMA,
    ],
    compiler_params=pltpu.CompilerParams(needs_layout_passes=False),
)
def _scatter_kernel(feat_hbm, src_hbm, dst_hbm, coef_hbm, out_hbm,
                    src_v, coef_v, rows_a, rows_b, dstb_a, dstb_b, acc,
                    gsem_a, gsem_b, dsem_a, dsem_b, ssem_a, ssem_b):
  c = lax.axis_index("c")
  s = lax.axis_index("s")
  gid = c * NS + s

  pltpu.sync_copy(src_hbm.at[pl.ds(gid * EPW, EPW)], src_v)
  pltpu.sync_copy(coef_hbm.at[pl.ds(gid * EPW, EPW)], coef_v)

  zeros16 = jnp.zeros((16,), _f32)

  def zrow(i, _):
    for v in range(D // 16):
      rows_a[i, pl.ds(v * 16, 16)] = zeros16
    return 0

  lax.fori_loop(0, K, zrow, 0)

  def zcopy(blk, _):
    pltpu.sync_copy(rows_a, acc.at[pl.ds(s * SLICE + blk * K, K)])
    return 0

  lax.fori_loop(0, SLICE // K, zcopy, 0)
  plsc.subcore_barrier()

  def fire_d(i, dstb, dsem):
    pltpu.async_copy(dst_hbm.at[pl.ds(gid * EPW + i * K, K)], dstb, dsem)

  def wait_d(i, dstb, dsem):
    pltpu.make_async_copy(dst_hbm.at[pl.ds(gid * EPW + i * K, K)], dstb,
                          dsem).wait()

  def fire_g(i, rows, gsem):
    pltpu.async_copy(feat_hbm.at[src_v.at[pl.ds(i * K, K)]], rows, gsem)

  def wait_g(i, rows, gsem):
    pltpu.make_async_copy(feat_hbm.at[src_v.at[pl.ds(i * K, K)]], rows,
                          gsem).wait()

  def fire_s(rows, dstb, ssem):
    pltpu.async_copy(rows, acc.at[dstb], ssem, add=True)

  def wait_s(rows, dstb, ssem):
    pltpu.make_async_copy(rows, acc.at[dstb], ssem).wait()

  def scale(i, rows):
    def ebody(j, _):
      for u in range(5):
        e = j * 5 + u
        ce = plsc.load_gather(coef_v, [jnp.full((16,), i * K + e, _i32)])
        for v in range(D // 16):
          vsl = pl.ds(v * 16, 16)
          rows[e, vsl] = rows[e, vsl] * ce
      return 0

    lax.fori_loop(0, K // 5, ebody, 0)

  buf = ((rows_a, dstb_a, gsem_a, dsem_a, ssem_a),
         (rows_b, dstb_b, gsem_b, dsem_b, ssem_b))

  def step(i, p, first=False, last=False):
    rows_p, dstb_p, gsem_p, dsem_p, ssem_p = buf[p]
    rows_q, dstb_q, gsem_q, dsem_q, ssem_q = buf[1 - p]
    if not first:
      wait_s(rows_q, dstb_q, ssem_q)
    if not last:
      fire_d(i + 1, dstb_q, dsem_q)
      fire_g(i + 1, rows_q, gsem_q)
    wait_g(i, rows_p, gsem_p)
    scale(i, rows_p)
    wait_d(i, dstb_p, dsem_p)
    fire_s(rows_p, dstb_p, ssem_p)

  fire_d(0, dstb_a, dsem_a)
  fire_g(0, rows_a, gsem_a)
  step(0, 0, first=True)

  def pair(t, _):
    step(2 * t + 1, 1)
    step(2 * t + 2, 0)
    return 0

  lax.fori_loop(0, (NSB - 3) // 2, pair, 0)
  step(NSB - 2, (NSB - 2) % 2)
  step(NSB - 1, (NSB - 1) % 2, last=True)
  wait_s(rows_a, dstb_a, ssem_a)

  plsc.subcore_barrier()
  pltpu.sync_copy(acc.at[pl.ds(s * SLICE, SLICE)],
                  out_hbm.at[c, pl.ds(s * SLICE, SLICE)])


def _post_body(p_ref, norms_ref, hs_ref, w_ref, b_ref, out_ref):
  agg = p_ref[0] + p_ref[1]
  h = lax.dot_general(
      agg, w_ref[...], (((1,), (1,)), ((), ())),
      preferred_element_type=_f32) + b_ref[...]
  in_col = jnp.transpose(norms_ref[1:2, :])
  h = h * in_col
  out_ref[...] = h[:N] + hs_ref[...]


_post_call = pl.pallas_call(
    _post_body,
    out_shape=jax.ShapeDtypeStruct((N, D), _f32),
)


def kernel(feature, edge_index, e_w, snorm_n, snorm_e, W_self, W, b):
  del snorm_n, snorm_e
  ew1 = e_w.reshape(E)
  src1 = edge_index[0]
  dst1 = edge_index[1]

  deg = _degree_kernel(src1, dst1)
  norms, h_s = _pre_call(deg, feature, W_self)
  coef = _coef_kernel(src1, ew1, norms[0])
  parts = _scatter_kernel(feature, src1, dst1, coef)
  h = _post_call(parts, norms, h_s, W, b.reshape(1, D))
  return (h, e_w)
